# baseline (speedup 1.0000x reference)
import functools

import jax
import jax.numpy as jnp
from jax import lax
from jax.experimental import pallas as pl
from jax.experimental.pallas import tpu as pltpu
from jax.experimental.pallas import tpu_sc as plsc

_L = 16
_NC = 2
_NS = 16
_NW = _NC * _NS

_N_ROWS = 1048576
_GROUP = 2 * 128
_N_ELEMS = _N_ROWS * 2
_PER_W = _N_ELEMS // _NW
_GROUPS_W = _PER_W // _GROUP

_mesh = plsc.VectorSubcoreMesh(core_axis_name="c", subcore_axis_name="s")


_NSLOTS = 8
_CHUNK = 4096
_PDIST = _NSLOTS // 2
_NCHUNKS = _PER_W // _CHUNK
_GROUPS_C = _CHUNK // _GROUP


@functools.partial(
    pl.kernel,
    mesh=_mesh,
    out_type=jax.ShapeDtypeStruct((_N_ELEMS,), jnp.float32),
    scratch_types=[pltpu.VMEM((_CHUNK,), jnp.float32)] * _NSLOTS
    + [pltpu.SemaphoreType.DMA] * (2 * _NSLOTS),
    compiler_params=pltpu.CompilerParams(needs_layout_passes=False),
)
def _hardmax_sc(x_hbm, out_hbm, *scratch):
    buf = scratch[:_NSLOTS]
    sin, sout = scratch[_NSLOTS:2 * _NSLOTS], scratch[2 * _NSLOTS:]
    wid = lax.axis_index("s") * _NC + lax.axis_index("c")
    base = wid * _PER_W

    def in_dma(c):
        s = c % _NSLOTS
        return pltpu.make_async_copy(
            x_hbm.at[pl.ds(base + c * _CHUNK, _CHUNK)], buf[s], sin[s])

    def out_dma(c):
        s = c % _NSLOTS
        return pltpu.make_async_copy(
            buf[s], out_hbm.at[pl.ds(base + c * _CHUNK, _CHUNK)], sout[s])

    def compute(s):
        bref = buf[s]

        def group_step(g, carry):
            goff = g * _GROUP
            for u in range(128 // _L):
                a_off = goff + u * _L
                b_off = a_off + 128
                a = bref[pl.ds(a_off, _L)]
                b = bref[pl.ds(b_off, _L)]
                ha = jnp.where(a >= b, 1.0, 0.0).astype(jnp.float32)
                hb = 1.0 - ha
                bref[pl.ds(a_off, _L)] = (ha - a) + a
                bref[pl.ds(b_off, _L)] = (hb - b) + b
            return carry

        lax.fori_loop(0, _GROUPS_C, group_step, 0)

    for p in range(_PDIST):
        in_dma(p).start()
    for c in range(_NCHUNKS):
        if c + _PDIST < _NCHUNKS:
            prev = c + _PDIST - _NSLOTS
            if prev >= 0:
                out_dma(prev).wait()
            in_dma(c + _PDIST).start()
        in_dma(c).wait()
        compute(c % _NSLOTS)
        out_dma(c).start()
    for c in range(max(0, _NCHUNKS - _NSLOTS + _PDIST), _NCHUNKS):
        out_dma(c).wait()


def kernel(x):
    n = x.shape[0]
    xg = x.reshape(n // 128, 128, 2).transpose(0, 2, 1)
    y_flat = _hardmax_sc(xg.reshape(-1))
    yg = y_flat.reshape(n // 128, 2, 128).transpose(0, 2, 1)
    return yg.reshape(n, 2)


# speedup vs baseline: 1.0311x; 1.0311x over previous
import functools

import jax
import jax.numpy as jnp
from jax import lax
from jax.experimental import pallas as pl
from jax.experimental.pallas import tpu as pltpu
from jax.experimental.pallas import tpu_sc as plsc

_L = 16
_NC = 2
_NS = 16
_NW = _NC * _NS

_N_ROWS = 1048576
_GROUP = 2 * 128
_N_ELEMS = _N_ROWS * 2
_PER_W = _N_ELEMS // _NW
_GROUPS_W = _PER_W // _GROUP

_mesh = plsc.VectorSubcoreMesh(core_axis_name="c", subcore_axis_name="s")


_NSLOTS = 4
_CHUNK = 16384
_PDIST = _NSLOTS // 2
_NCHUNKS = _PER_W // _CHUNK
_GROUPS_C = _CHUNK // _GROUP


@functools.partial(
    pl.kernel,
    mesh=_mesh,
    out_type=jax.ShapeDtypeStruct((_N_ELEMS,), jnp.float32),
    scratch_types=[pltpu.VMEM((_CHUNK,), jnp.float32)] * _NSLOTS
    + [pltpu.SemaphoreType.DMA] * (2 * _NSLOTS),
    compiler_params=pltpu.CompilerParams(needs_layout_passes=False),
)
def _hardmax_sc(x_hbm, out_hbm, *scratch):
    buf = scratch[:_NSLOTS]
    sin, sout = scratch[_NSLOTS:2 * _NSLOTS], scratch[2 * _NSLOTS:]
    wid = lax.axis_index("s") * _NC + lax.axis_index("c")
    base = wid * _PER_W

    def in_dma(c):
        s = c % _NSLOTS
        return pltpu.make_async_copy(
            x_hbm.at[pl.ds(base + c * _CHUNK, _CHUNK)], buf[s], sin[s])

    def out_dma(c):
        s = c % _NSLOTS
        return pltpu.make_async_copy(
            buf[s], out_hbm.at[pl.ds(base + c * _CHUNK, _CHUNK)], sout[s])

    def compute(s):
        bref = buf[s]

        def group_step(g, carry):
            goff = g * _GROUP
            for u in range(128 // _L):
                a_off = goff + u * _L
                b_off = a_off + 128
                a = bref[pl.ds(a_off, _L)]
                b = bref[pl.ds(b_off, _L)]
                ha = jnp.where(a >= b, 1.0, 0.0).astype(jnp.float32)
                hb = 1.0 - ha
                bref[pl.ds(a_off, _L)] = (ha - a) + a
                bref[pl.ds(b_off, _L)] = (hb - b) + b
            return carry

        lax.fori_loop(0, _GROUPS_C, group_step, 0)

    for p in range(_PDIST):
        in_dma(p).start()
    for c in range(_NCHUNKS):
        if c + _PDIST < _NCHUNKS:
            prev = c + _PDIST - _NSLOTS
            if prev >= 0:
                out_dma(prev).wait()
            in_dma(c + _PDIST).start()
        in_dma(c).wait()
        compute(c % _NSLOTS)
        out_dma(c).start()
    for c in range(max(0, _NCHUNKS - _NSLOTS + _PDIST), _NCHUNKS):
        out_dma(c).wait()


def kernel(x):
    n = x.shape[0]
    xg = x.reshape(n // 128, 128, 2).transpose(0, 2, 1)
    y_flat = _hardmax_sc(xg.reshape(-1))
    yg = y_flat.reshape(n // 128, 2, 128).transpose(0, 2, 1)
    return yg.reshape(n, 2)
